# baseline (speedup 1.0000x reference)
import functools

import jax
import jax.numpy as jnp
from jax import lax
from jax.experimental import pallas as pl
from jax.experimental.pallas import tpu as pltpu
from jax.experimental.pallas import tpu_sc as plsc

N = 10000
D = 128
H = 128
G = 128
NC = 2
NS = 16
CHUNK = 128
NB = 4
SIDX = 32
NIB = 5
CT = SIDX * NIB
E_PAD = CT * NS * CHUNK
HF = 64
N_PAD = 10112
NBLK = N_PAD // 128
ROWS_PER_TILE = N_PAD // NS

_mesh = lambda: plsc.VectorSubcoreMesh(core_axis_name="c", subcore_axis_name="s")


def _sc_degrees(dsts, onehot, zeros16):
    @functools.partial(
        pl.kernel,
        out_type=jax.ShapeDtypeStruct((NC, N_PAD, 16), jnp.float32),
        mesh=_mesh(),
        scratch_types=[
            pltpu.VMEM((CT, CHUNK), jnp.int32),
            pltpu.VMEM((CHUNK, 16), jnp.float32),
            pltpu.VMEM_SHARED((N_PAD, 16), jnp.float32),
        ],
    )
    def k(dst_hbm, oh_hbm, z_hbm, out_hbm, dst_v, oh_v, cnt_sh):
        c = lax.axis_index("c")
        s = lax.axis_index("s")
        r0 = s * ROWS_PER_TILE
        pltpu.sync_copy(z_hbm.at[pl.ds(r0, ROWS_PER_TILE)],
                        cnt_sh.at[pl.ds(r0, ROWS_PER_TILE)])
        pltpu.sync_copy(dst_hbm.at[c, s], dst_v)
        pltpu.sync_copy(oh_hbm, oh_v)
        plsc.subcore_barrier()

        def body(j, carry):
            pltpu.sync_copy(oh_v, cnt_sh.at[dst_v.at[j]], add=True)
            return carry

        lax.fori_loop(0, CT, body, 0)
        plsc.subcore_barrier()
        pltpu.sync_copy(cnt_sh.at[pl.ds(r0, ROWS_PER_TILE)],
                        out_hbm.at[c, pl.ds(r0, ROWS_PER_TILE)])

    return k(dsts, onehot, zeros16)


def _sc_conv(g_lo, g_hi, srcs, dsts, zeros_hf):
    @functools.partial(
        pl.kernel,
        out_type=[jax.ShapeDtypeStruct((NC, N_PAD, HF), jnp.float32),
                  jax.ShapeDtypeStruct((NC, N_PAD, HF), jnp.float32)],
        mesh=_mesh(),
        scratch_types=[
            pltpu.VMEM((2, SIDX, CHUNK), jnp.int32),
            pltpu.VMEM((2, SIDX, CHUNK), jnp.int32),
            pltpu.VMEM((NB, CHUNK, HF), jnp.float32),
            pltpu.VMEM_SHARED((N_PAD, HF), jnp.float32),
            pltpu.VMEM_SHARED((N_PAD, HF), jnp.float32),
            [pltpu.SemaphoreType.DMA] * NB,
            [pltpu.SemaphoreType.DMA] * NB,
            pltpu.SemaphoreType.DMA,
        ],
        compiler_params=pltpu.CompilerParams(use_tc_tiling_on_sc=False),
    )
    def k(glo_hbm, ghi_hbm, src_hbm, dst_hbm, z_hbm, out_lo, out_hi,
          src_v, dst_v, rows_v, tbl_sh, agg_sh, sems, ssems, sem_i):
        c = lax.axis_index("c")
        s = lax.axis_index("s")
        r0 = s * ROWS_PER_TILE

        def idx_load(bi, slot):
            pltpu.async_copy(src_hbm.at[c, s, pl.ds(bi * SIDX, SIDX)],
                             src_v.at[slot], sem_i)
            pltpu.async_copy(dst_hbm.at[c, s, pl.ds(bi * SIDX, SIDX)],
                             dst_v.at[slot], sem_i)

        def idx_wait(bi, slot):
            pltpu.make_async_copy(src_hbm.at[c, s, pl.ds(bi * SIDX, SIDX)],
                                  src_v.at[slot], sem_i).wait()
            pltpu.make_async_copy(dst_hbm.at[c, s, pl.ds(bi * SIDX, SIDX)],
                                  dst_v.at[slot], sem_i).wait()

        for tbl_hbm, o_hbm in ((glo_hbm, out_lo), (ghi_hbm, out_hi)):
            pltpu.sync_copy(z_hbm.at[pl.ds(r0, ROWS_PER_TILE)],
                            agg_sh.at[pl.ds(r0, ROWS_PER_TILE)])
            pltpu.sync_copy(tbl_hbm.at[c, pl.ds(r0, ROWS_PER_TILE)],
                            tbl_sh.at[pl.ds(r0, ROWS_PER_TILE)])
            idx_load(0, 0)
            plsc.subcore_barrier()

            def blk(bi, carry):
                slot = lax.rem(bi, 2)
                idx_wait(bi, slot)

                @pl.when(bi + 1 < NIB)
                def _():
                    idx_load(bi + 1, 1 - slot)

                src_b = src_v.at[slot]
                dst_b = dst_v.at[slot]
                for p in range(NB - 1):
                    pltpu.async_copy(tbl_sh.at[src_b.at[p]], rows_v.at[p],
                                     sems[p])
                for j in range(SIDX):
                    b = j % NB
                    pltpu.make_async_copy(tbl_sh.at[src_b.at[j]], rows_v.at[b],
                                          sems[b]).wait()
                    pltpu.async_copy(rows_v.at[b], agg_sh.at[dst_b.at[j]],
                                     ssems[b], add=True)
                    nxt = j + NB - 1
                    if nxt < SIDX:
                        nb = nxt % NB
                        if nxt >= NB:
                            pltpu.make_async_copy(
                                rows_v.at[nb], agg_sh.at[dst_b.at[nxt - NB]],
                                ssems[nb]).wait()
                        pltpu.async_copy(tbl_sh.at[src_b.at[nxt]],
                                         rows_v.at[nb], sems[nb])
                for j in range(SIDX - NB, SIDX):
                    b = j % NB
                    pltpu.make_async_copy(rows_v.at[b],
                                          agg_sh.at[dst_b.at[j]],
                                          ssems[b]).wait()
                return carry

            lax.fori_loop(0, NIB, blk, 0)
            plsc.subcore_barrier()
            pltpu.sync_copy(agg_sh.at[pl.ds(r0, ROWS_PER_TILE)],
                            o_hbm.at[c, pl.ds(r0, ROWS_PER_TILE)])
            plsc.subcore_barrier()

    return k(g_lo, g_hi, srcs, dsts, zeros_hf)


def _prelu(x, a):
    return jnp.where(x > 0, x, a * x)


def _full(shape):
    nd = len(shape)
    return pl.BlockSpec(shape, lambda *_, _n=nd: (0,) * _n)


def _tc_pre(x_pad, cnt, W1, b1):
    def body(x_ref, cnt_ref, w_ref, b_ref, h_ref, glo_ref, ghi_ref):
        i = pl.program_id(0)
        xb = x_ref[...]
        row = i * 128 + lax.broadcasted_iota(jnp.int32, (128, 1), 0)
        valid = row < N
        for t in range(2):
            h = jnp.dot(xb, w_ref[t], preferred_element_type=jnp.float32) + b_ref[t]
            deg = cnt_ref[t, :, 0:1] + 1.0
            dinv = lax.rsqrt(deg)
            h_ref[t] = h
            g = jnp.where(valid, dinv * h, 0.0)
            glo_ref[t] = g[:, :HF]
            ghi_ref[t] = g[:, HF:]

    return pl.pallas_call(
        body,
        grid=(NBLK,),
        in_specs=[
            pl.BlockSpec((128, 128), lambda i: (i, 0)),
            pl.BlockSpec((2, 128, 16), lambda i: (0, i, 0)),
            _full((2, 128, 128)),
            _full((2, 128)),
        ],
        out_specs=[
            pl.BlockSpec((2, 128, 128), lambda i: (0, i, 0)),
            pl.BlockSpec((2, 128, HF), lambda i: (0, i, 0)),
            pl.BlockSpec((2, 128, HF), lambda i: (0, i, 0)),
        ],
        out_shape=[
            jax.ShapeDtypeStruct((2, N_PAD, 128), jnp.float32),
            jax.ShapeDtypeStruct((2, N_PAD, HF), jnp.float32),
            jax.ShapeDtypeStruct((2, N_PAD, HF), jnp.float32),
        ],
    )(x_pad, cnt, W1, b1)


def _tc_mid(agg_lo, agg_hi, h1, cnt, batch3, W2, b2, a1):
    def body(alo_ref, ahi_ref, h1_ref, cnt_ref, bat_ref, w_ref, b_ref, a_ref,
             h2_ref, g2lo_ref, g2hi_ref, pool_ref):
        i = pl.program_id(0)
        row = i * 128 + lax.broadcasted_iota(jnp.int32, (128, 1), 0)
        valid = row < N
        bb = bat_ref[0]
        mt = (lax.broadcasted_iota(jnp.int32, (G, 128), 0) == bb
              ).astype(jnp.float32)

        @pl.when(i == 0)
        def _():
            pool_ref[...] = jnp.zeros_like(pool_ref)

        for t in range(2):
            deg = cnt_ref[t, :, 0:1] + 1.0
            dinv = lax.rsqrt(deg)
            agg = jnp.concatenate([alo_ref[t], ahi_ref[t]], axis=1)
            p1 = agg * dinv + h1_ref[t] * (1.0 / deg)
            p1 = _prelu(p1, a_ref[t])
            pool_ref[t] += lax.dot_general(
                mt, p1, (((1,), (0,)), ((), ())),
                preferred_element_type=jnp.float32)
            h2 = jnp.dot(p1, w_ref[t], preferred_element_type=jnp.float32) + b_ref[t]
            h2_ref[t] = h2
            g2 = jnp.where(valid, dinv * h2, 0.0)
            g2lo_ref[t] = g2[:, :HF]
            g2hi_ref[t] = g2[:, HF:]

    return pl.pallas_call(
        body,
        grid=(NBLK,),
        in_specs=[
            pl.BlockSpec((2, 128, HF), lambda i: (0, i, 0)),
            pl.BlockSpec((2, 128, HF), lambda i: (0, i, 0)),
            pl.BlockSpec((2, 128, 128), lambda i: (0, i, 0)),
            pl.BlockSpec((2, 128, 16), lambda i: (0, i, 0)),
            pl.BlockSpec((1, 1, 128), lambda i: (i, 0, 0)),
            _full((2, 128, 128)),
            _full((2, 128)),
            pl.BlockSpec(memory_space=pltpu.SMEM),
        ],
        out_specs=[
            pl.BlockSpec((2, 128, 128), lambda i: (0, i, 0)),
            pl.BlockSpec((2, 128, HF), lambda i: (0, i, 0)),
            pl.BlockSpec((2, 128, HF), lambda i: (0, i, 0)),
            pl.BlockSpec((2, G, 128), lambda i: (0, 0, 0)),
        ],
        out_shape=[
            jax.ShapeDtypeStruct((2, N_PAD, 128), jnp.float32),
            jax.ShapeDtypeStruct((2, N_PAD, HF), jnp.float32),
            jax.ShapeDtypeStruct((2, N_PAD, HF), jnp.float32),
            jax.ShapeDtypeStruct((2, G, 128), jnp.float32),
        ],
    )(agg_lo, agg_hi, h1, cnt, batch3, W2, b2, a1)


def _tc_post(agg2_lo, agg2_hi, h2, cnt, batch3, mW, mb, scal,
             pool1, gw0, gw12, gws, gb):
    def body(alo_ref, ahi_ref, h2_ref, cnt_ref, bat_ref, w_ref, b_ref, a_ref,
             p1_ref, gw0_ref, gw12_ref, gws_ref, gb_ref,
             lv1_ref, lv2_ref, pool_ref, gv1_ref, gv2_ref):
        i = pl.program_id(0)
        bb = bat_ref[0]
        mt = (lax.broadcasted_iota(jnp.int32, (G, 128), 0) == bb
              ).astype(jnp.float32)

        @pl.when(i == 0)
        def _():
            pool_ref[...] = jnp.zeros_like(pool_ref)

        outs = (lv1_ref, lv2_ref)
        for t in range(2):
            deg = cnt_ref[t, :, 0:1] + 1.0
            dinv = lax.rsqrt(deg)
            agg = jnp.concatenate([alo_ref[t], ahi_ref[t]], axis=1)
            p2 = agg * dinv + h2_ref[t] * (1.0 / deg)
            p2 = _prelu(p2, a_ref[t])
            pool_ref[t] += lax.dot_general(
                mt, p2, (((1,), (0,)), ((), ())),
                preferred_element_type=jnp.float32)
            hcur = p2
            for layer in range(3):
                hcur = _prelu(
                    jnp.dot(hcur, w_ref[layer],
                            preferred_element_type=jnp.float32) + b_ref[layer],
                    a_ref[2 + layer])
            outs[t][...] = hcur + jnp.dot(
                p2, w_ref[3], preferred_element_type=jnp.float32) + b_ref[3]

        @pl.when(i == NBLK - 1)
        def _():
            gouts = (gv1_ref, gv2_ref)
            for t in range(2):
                gp = jnp.concatenate([p1_ref[t], pool_ref[t]], axis=1)
                gcur = _prelu(
                    jnp.dot(gp, gw0_ref[...],
                            preferred_element_type=jnp.float32) + gb_ref[0],
                    a_ref[5])
                for layer in range(2):
                    gcur = _prelu(
                        jnp.dot(gcur, gw12_ref[layer],
                                preferred_element_type=jnp.float32)
                        + gb_ref[1 + layer], a_ref[6 + layer])
                gouts[t][...] = gcur + jnp.dot(
                    gp, gws_ref[...],
                    preferred_element_type=jnp.float32) + gb_ref[3]

    return pl.pallas_call(
        body,
        grid=(NBLK,),
        in_specs=[
            pl.BlockSpec((2, 128, HF), lambda i: (0, i, 0)),
            pl.BlockSpec((2, 128, HF), lambda i: (0, i, 0)),
            pl.BlockSpec((2, 128, 128), lambda i: (0, i, 0)),
            pl.BlockSpec((2, 128, 16), lambda i: (0, i, 0)),
            pl.BlockSpec((1, 1, 128), lambda i: (i, 0, 0)),
            _full((4, 128, 128)),
            _full((4, 128)),
            pl.BlockSpec(memory_space=pltpu.SMEM),
            _full((2, G, 128)),
            _full((256, 128)),
            _full((2, 128, 128)),
            _full((256, 128)),
            _full((4, 128)),
        ],
        out_specs=[
            pl.BlockSpec((128, 128), lambda i: (i, 0)),
            pl.BlockSpec((128, 128), lambda i: (i, 0)),
            pl.BlockSpec((2, G, 128), lambda i: (0, 0, 0)),
            pl.BlockSpec((G, 128), lambda i: (0, 0)),
            pl.BlockSpec((G, 128), lambda i: (0, 0)),
        ],
        out_shape=[
            jax.ShapeDtypeStruct((N_PAD, 128), jnp.float32),
            jax.ShapeDtypeStruct((N_PAD, 128), jnp.float32),
            jax.ShapeDtypeStruct((2, G, 128), jnp.float32),
            jax.ShapeDtypeStruct((G, 128), jnp.float32),
            jax.ShapeDtypeStruct((G, 128), jnp.float32),
        ],
    )(agg2_lo, agg2_hi, h2, cnt, batch3, mW, mb, scal,
      pool1, gw0, gw12, gws, gb)


def kernel(x, edge_index, diff_index, batch, gnn1, gnn2, mlp1, mlp2):
    i32 = jnp.int32
    E = edge_index.shape[1]
    pad = E_PAD - E

    srcs = jnp.stack([
        jnp.concatenate([edge_index[0], jnp.full((pad,), N, i32)]),
        jnp.concatenate([diff_index[0], jnp.full((pad,), N, i32)]),
    ]).reshape(NC, NS, CT, CHUNK)
    dsts = jnp.stack([
        jnp.concatenate([edge_index[1], jnp.full((pad,), N, i32)]),
        jnp.concatenate([diff_index[1], jnp.full((pad,), N, i32)]),
    ]).reshape(NC, NS, CT, CHUNK)

    onehot = jnp.concatenate(
        [jnp.ones((CHUNK, 1), jnp.float32), jnp.zeros((CHUNK, 15), jnp.float32)],
        axis=1)
    zeros16 = jnp.zeros((N_PAD, 16), jnp.float32)
    zeros_hf = jnp.zeros((N_PAD, HF), jnp.float32)

    x_pad = jnp.concatenate([x, jnp.zeros((N_PAD - N, D), jnp.float32)])
    batch3 = jnp.concatenate([batch, jnp.full((N_PAD - N,), G, i32)]
                             ).reshape(NBLK, 1, 128)

    W1 = jnp.stack([gnn1['W0'], gnn2['W0']])
    b1 = jnp.stack([gnn1['b0'], gnn2['b0']])
    W2 = jnp.stack([gnn1['W1'], gnn2['W1']])
    b2 = jnp.stack([gnn1['b1'], gnn2['b1']])
    a1 = jnp.stack([gnn1['a0'], gnn2['a0']])
    mW = jnp.stack([mlp1['W0'], mlp1['W1'], mlp1['W2'], mlp1['Ws']])
    mb = jnp.stack([mlp1['b0'], mlp1['b1'], mlp1['b2'], mlp1['bs']])
    pscal = jnp.stack([gnn1['a1'], gnn2['a1'],
                       mlp1['a0'], mlp1['a1'], mlp1['a2'],
                       mlp2['a0'], mlp2['a1'], mlp2['a2']])
    gb = jnp.stack([mlp2['b0'], mlp2['b1'], mlp2['b2'], mlp2['bs']])
    gw12 = jnp.stack([mlp2['W1'], mlp2['W2']])

    cnt = _sc_degrees(dsts, onehot, zeros16)
    h1, g1lo, g1hi = _tc_pre(x_pad, cnt, W1, b1)
    agg1_lo, agg1_hi = _sc_conv(g1lo, g1hi, srcs, dsts, zeros_hf)
    h2, g2lo, g2hi, pool1 = _tc_mid(agg1_lo, agg1_hi, h1, cnt, batch3,
                                    W2, b2, a1)
    agg2_lo, agg2_hi = _sc_conv(g2lo, g2hi, srcs, dsts, zeros_hf)
    lv1, lv2, _, gv1, gv2 = _tc_post(agg2_lo, agg2_hi, h2, cnt, batch3,
                                     mW, mb, pscal,
                                     pool1, mlp2['W0'], gw12, mlp2['Ws'], gb)

    return (lv1[:N], gv1, lv2[:N], gv2)


# speedup vs baseline: 1.0076x; 1.0076x over previous
import functools

import jax
import jax.numpy as jnp
from jax import lax
from jax.experimental import pallas as pl
from jax.experimental.pallas import tpu as pltpu
from jax.experimental.pallas import tpu_sc as plsc

N = 10000
D = 128
H = 128
G = 128
NC = 2
NS = 16
CHUNK = 128
NB = 3
SIDX = 40
NIB = 4
CT = SIDX * NIB
E_PAD = CT * NS * CHUNK
HF = 64
N_PAD = 10112
NBLK = N_PAD // 128
ROWS_PER_TILE = N_PAD // NS

_mesh = lambda: plsc.VectorSubcoreMesh(core_axis_name="c", subcore_axis_name="s")


def _sc_degrees(dsts, onehot, zeros16):
    @functools.partial(
        pl.kernel,
        out_type=jax.ShapeDtypeStruct((NC, N_PAD, 16), jnp.float32),
        mesh=_mesh(),
        scratch_types=[
            pltpu.VMEM((CT, CHUNK), jnp.int32),
            pltpu.VMEM((CHUNK, 16), jnp.float32),
            pltpu.VMEM_SHARED((N_PAD, 16), jnp.float32),
            pltpu.SemaphoreType.DMA,
        ],
    )
    def k(dst_hbm, oh_hbm, z_hbm, out_hbm, dst_v, oh_v, cnt_sh, sem_d):
        c = lax.axis_index("c")
        s = lax.axis_index("s")
        r0 = s * ROWS_PER_TILE
        pltpu.sync_copy(z_hbm.at[pl.ds(r0, ROWS_PER_TILE)],
                        cnt_sh.at[pl.ds(r0, ROWS_PER_TILE)])
        pltpu.sync_copy(dst_hbm.at[c, s], dst_v)
        pltpu.sync_copy(oh_hbm, oh_v)
        plsc.subcore_barrier()

        def body(j, carry):
            pltpu.async_copy(oh_v, cnt_sh.at[dst_v.at[j]], sem_d, add=True)
            return carry

        lax.fori_loop(0, CT, body, 0)

        def drain(j, carry):
            pltpu.make_async_copy(oh_v, cnt_sh.at[dst_v.at[j]], sem_d).wait()
            return carry

        lax.fori_loop(0, CT, drain, 0)
        plsc.subcore_barrier()
        pltpu.sync_copy(cnt_sh.at[pl.ds(r0, ROWS_PER_TILE)],
                        out_hbm.at[c, pl.ds(r0, ROWS_PER_TILE)])

    return k(dsts, onehot, zeros16)


def _sc_conv(g_lo, g_hi, srcs, dsts, zeros_hf):
    @functools.partial(
        pl.kernel,
        out_type=[jax.ShapeDtypeStruct((NC, N_PAD, HF), jnp.float32),
                  jax.ShapeDtypeStruct((NC, N_PAD, HF), jnp.float32)],
        mesh=_mesh(),
        scratch_types=[
            pltpu.VMEM((2, SIDX, CHUNK), jnp.int32),
            pltpu.VMEM((2, SIDX, CHUNK), jnp.int32),
            pltpu.VMEM((NB, CHUNK, HF), jnp.float32),
            pltpu.VMEM_SHARED((N_PAD, HF), jnp.float32),
            pltpu.VMEM_SHARED((N_PAD, HF), jnp.float32),
            [pltpu.SemaphoreType.DMA] * NB,
            [pltpu.SemaphoreType.DMA] * NB,
            pltpu.SemaphoreType.DMA,
        ],
        compiler_params=pltpu.CompilerParams(use_tc_tiling_on_sc=False),
    )
    def k(glo_hbm, ghi_hbm, src_hbm, dst_hbm, z_hbm, out_lo, out_hi,
          src_v, dst_v, rows_v, tbl_sh, agg_sh, sems, ssems, sem_i):
        c = lax.axis_index("c")
        s = lax.axis_index("s")
        r0 = s * ROWS_PER_TILE

        def idx_load(bi, slot):
            pltpu.async_copy(src_hbm.at[c, s, pl.ds(bi * SIDX, SIDX)],
                             src_v.at[slot], sem_i)
            pltpu.async_copy(dst_hbm.at[c, s, pl.ds(bi * SIDX, SIDX)],
                             dst_v.at[slot], sem_i)

        def idx_wait(bi, slot):
            pltpu.make_async_copy(src_hbm.at[c, s, pl.ds(bi * SIDX, SIDX)],
                                  src_v.at[slot], sem_i).wait()
            pltpu.make_async_copy(dst_hbm.at[c, s, pl.ds(bi * SIDX, SIDX)],
                                  dst_v.at[slot], sem_i).wait()

        for tbl_hbm, o_hbm in ((glo_hbm, out_lo), (ghi_hbm, out_hi)):
            pltpu.sync_copy(z_hbm.at[pl.ds(r0, ROWS_PER_TILE)],
                            agg_sh.at[pl.ds(r0, ROWS_PER_TILE)])
            pltpu.sync_copy(tbl_hbm.at[c, pl.ds(r0, ROWS_PER_TILE)],
                            tbl_sh.at[pl.ds(r0, ROWS_PER_TILE)])
            idx_load(0, 0)
            plsc.subcore_barrier()

            def blk(bi, carry):
                slot = lax.rem(bi, 2)
                idx_wait(bi, slot)

                @pl.when(bi + 1 < NIB)
                def _():
                    idx_load(bi + 1, 1 - slot)

                src_b = src_v.at[slot]
                dst_b = dst_v.at[slot]
                for p in range(NB - 1):
                    pltpu.async_copy(tbl_sh.at[src_b.at[p]], rows_v.at[p],
                                     sems[p])
                for j in range(SIDX):
                    b = j % NB
                    pltpu.make_async_copy(tbl_sh.at[src_b.at[j]], rows_v.at[b],
                                          sems[b]).wait()
                    pltpu.async_copy(rows_v.at[b], agg_sh.at[dst_b.at[j]],
                                     ssems[b], add=True)
                    nxt = j + NB - 1
                    if nxt < SIDX:
                        nb = nxt % NB
                        if nxt >= NB:
                            pltpu.make_async_copy(
                                rows_v.at[nb], agg_sh.at[dst_b.at[nxt - NB]],
                                ssems[nb]).wait()
                        pltpu.async_copy(tbl_sh.at[src_b.at[nxt]],
                                         rows_v.at[nb], sems[nb])
                for j in range(SIDX - NB, SIDX):
                    b = j % NB
                    pltpu.make_async_copy(rows_v.at[b],
                                          agg_sh.at[dst_b.at[j]],
                                          ssems[b]).wait()
                return carry

            lax.fori_loop(0, NIB, blk, 0)
            plsc.subcore_barrier()
            pltpu.sync_copy(agg_sh.at[pl.ds(r0, ROWS_PER_TILE)],
                            o_hbm.at[c, pl.ds(r0, ROWS_PER_TILE)])
            plsc.subcore_barrier()

    return k(g_lo, g_hi, srcs, dsts, zeros_hf)


def _prelu(x, a):
    return jnp.where(x > 0, x, a * x)


def _full(shape):
    nd = len(shape)
    return pl.BlockSpec(shape, lambda *_, _n=nd: (0,) * _n)


def _tc_pre(x_pad, cnt, W1, b1):
    def body(x_ref, cnt_ref, w_ref, b_ref, h_ref, glo_ref, ghi_ref):
        i = pl.program_id(0)
        xb = x_ref[...]
        row = i * 128 + lax.broadcasted_iota(jnp.int32, (128, 1), 0)
        valid = row < N
        for t in range(2):
            h = jnp.dot(xb, w_ref[t], preferred_element_type=jnp.float32) + b_ref[t]
            deg = cnt_ref[t, :, 0:1] + 1.0
            dinv = lax.rsqrt(deg)
            h_ref[t] = h
            g = jnp.where(valid, dinv * h, 0.0)
            glo_ref[t] = g[:, :HF]
            ghi_ref[t] = g[:, HF:]

    return pl.pallas_call(
        body,
        grid=(NBLK,),
        in_specs=[
            pl.BlockSpec((128, 128), lambda i: (i, 0)),
            pl.BlockSpec((2, 128, 16), lambda i: (0, i, 0)),
            _full((2, 128, 128)),
            _full((2, 128)),
        ],
        out_specs=[
            pl.BlockSpec((2, 128, 128), lambda i: (0, i, 0)),
            pl.BlockSpec((2, 128, HF), lambda i: (0, i, 0)),
            pl.BlockSpec((2, 128, HF), lambda i: (0, i, 0)),
        ],
        out_shape=[
            jax.ShapeDtypeStruct((2, N_PAD, 128), jnp.float32),
            jax.ShapeDtypeStruct((2, N_PAD, HF), jnp.float32),
            jax.ShapeDtypeStruct((2, N_PAD, HF), jnp.float32),
        ],
    )(x_pad, cnt, W1, b1)


def _tc_mid(agg_lo, agg_hi, h1, cnt, batch3, W2, b2, a1):
    def body(alo_ref, ahi_ref, h1_ref, cnt_ref, bat_ref, w_ref, b_ref, a_ref,
             h2_ref, g2lo_ref, g2hi_ref, pool_ref):
        i = pl.program_id(0)
        row = i * 128 + lax.broadcasted_iota(jnp.int32, (128, 1), 0)
        valid = row < N
        bb = bat_ref[0]
        mt = (lax.broadcasted_iota(jnp.int32, (G, 128), 0) == bb
              ).astype(jnp.float32)

        @pl.when(i == 0)
        def _():
            pool_ref[...] = jnp.zeros_like(pool_ref)

        for t in range(2):
            deg = cnt_ref[t, :, 0:1] + 1.0
            dinv = lax.rsqrt(deg)
            agg = jnp.concatenate([alo_ref[t], ahi_ref[t]], axis=1)
            p1 = agg * dinv + h1_ref[t] * (1.0 / deg)
            p1 = _prelu(p1, a_ref[t])
            pool_ref[t] += lax.dot_general(
                mt, p1, (((1,), (0,)), ((), ())),
                preferred_element_type=jnp.float32)
            h2 = jnp.dot(p1, w_ref[t], preferred_element_type=jnp.float32) + b_ref[t]
            h2_ref[t] = h2
            g2 = jnp.where(valid, dinv * h2, 0.0)
            g2lo_ref[t] = g2[:, :HF]
            g2hi_ref[t] = g2[:, HF:]

    return pl.pallas_call(
        body,
        grid=(NBLK,),
        in_specs=[
            pl.BlockSpec((2, 128, HF), lambda i: (0, i, 0)),
            pl.BlockSpec((2, 128, HF), lambda i: (0, i, 0)),
            pl.BlockSpec((2, 128, 128), lambda i: (0, i, 0)),
            pl.BlockSpec((2, 128, 16), lambda i: (0, i, 0)),
            pl.BlockSpec((1, 1, 128), lambda i: (i, 0, 0)),
            _full((2, 128, 128)),
            _full((2, 128)),
            pl.BlockSpec(memory_space=pltpu.SMEM),
        ],
        out_specs=[
            pl.BlockSpec((2, 128, 128), lambda i: (0, i, 0)),
            pl.BlockSpec((2, 128, HF), lambda i: (0, i, 0)),
            pl.BlockSpec((2, 128, HF), lambda i: (0, i, 0)),
            pl.BlockSpec((2, G, 128), lambda i: (0, 0, 0)),
        ],
        out_shape=[
            jax.ShapeDtypeStruct((2, N_PAD, 128), jnp.float32),
            jax.ShapeDtypeStruct((2, N_PAD, HF), jnp.float32),
            jax.ShapeDtypeStruct((2, N_PAD, HF), jnp.float32),
            jax.ShapeDtypeStruct((2, G, 128), jnp.float32),
        ],
    )(agg_lo, agg_hi, h1, cnt, batch3, W2, b2, a1)


def _tc_post(agg2_lo, agg2_hi, h2, cnt, batch3, mW, mb, scal,
             pool1, gw0, gw12, gws, gb):
    def body(alo_ref, ahi_ref, h2_ref, cnt_ref, bat_ref, w_ref, b_ref, a_ref,
             p1_ref, gw0_ref, gw12_ref, gws_ref, gb_ref,
             lv1_ref, lv2_ref, pool_ref, gv1_ref, gv2_ref):
        i = pl.program_id(0)
        bb = bat_ref[0]
        mt = (lax.broadcasted_iota(jnp.int32, (G, 128), 0) == bb
              ).astype(jnp.float32)

        @pl.when(i == 0)
        def _():
            pool_ref[...] = jnp.zeros_like(pool_ref)

        outs = (lv1_ref, lv2_ref)
        for t in range(2):
            deg = cnt_ref[t, :, 0:1] + 1.0
            dinv = lax.rsqrt(deg)
            agg = jnp.concatenate([alo_ref[t], ahi_ref[t]], axis=1)
            p2 = agg * dinv + h2_ref[t] * (1.0 / deg)
            p2 = _prelu(p2, a_ref[t])
            pool_ref[t] += lax.dot_general(
                mt, p2, (((1,), (0,)), ((), ())),
                preferred_element_type=jnp.float32)
            hcur = p2
            for layer in range(3):
                hcur = _prelu(
                    jnp.dot(hcur, w_ref[layer],
                            preferred_element_type=jnp.float32) + b_ref[layer],
                    a_ref[2 + layer])
            outs[t][...] = hcur + jnp.dot(
                p2, w_ref[3], preferred_element_type=jnp.float32) + b_ref[3]

        @pl.when(i == NBLK - 1)
        def _():
            gouts = (gv1_ref, gv2_ref)
            for t in range(2):
                gp = jnp.concatenate([p1_ref[t], pool_ref[t]], axis=1)
                gcur = _prelu(
                    jnp.dot(gp, gw0_ref[...],
                            preferred_element_type=jnp.float32) + gb_ref[0],
                    a_ref[5])
                for layer in range(2):
                    gcur = _prelu(
                        jnp.dot(gcur, gw12_ref[layer],
                                preferred_element_type=jnp.float32)
                        + gb_ref[1 + layer], a_ref[6 + layer])
                gouts[t][...] = gcur + jnp.dot(
                    gp, gws_ref[...],
                    preferred_element_type=jnp.float32) + gb_ref[3]

    return pl.pallas_call(
        body,
        grid=(NBLK,),
        in_specs=[
            pl.BlockSpec((2, 128, HF), lambda i: (0, i, 0)),
            pl.BlockSpec((2, 128, HF), lambda i: (0, i, 0)),
            pl.BlockSpec((2, 128, 128), lambda i: (0, i, 0)),
            pl.BlockSpec((2, 128, 16), lambda i: (0, i, 0)),
            pl.BlockSpec((1, 1, 128), lambda i: (i, 0, 0)),
            _full((4, 128, 128)),
            _full((4, 128)),
            pl.BlockSpec(memory_space=pltpu.SMEM),
            _full((2, G, 128)),
            _full((256, 128)),
            _full((2, 128, 128)),
            _full((256, 128)),
            _full((4, 128)),
        ],
        out_specs=[
            pl.BlockSpec((128, 128), lambda i: (i, 0)),
            pl.BlockSpec((128, 128), lambda i: (i, 0)),
            pl.BlockSpec((2, G, 128), lambda i: (0, 0, 0)),
            pl.BlockSpec((G, 128), lambda i: (0, 0)),
            pl.BlockSpec((G, 128), lambda i: (0, 0)),
        ],
        out_shape=[
            jax.ShapeDtypeStruct((N_PAD, 128), jnp.float32),
            jax.ShapeDtypeStruct((N_PAD, 128), jnp.float32),
            jax.ShapeDtypeStruct((2, G, 128), jnp.float32),
            jax.ShapeDtypeStruct((G, 128), jnp.float32),
            jax.ShapeDtypeStruct((G, 128), jnp.float32),
        ],
    )(agg2_lo, agg2_hi, h2, cnt, batch3, mW, mb, scal,
      pool1, gw0, gw12, gws, gb)


def kernel(x, edge_index, diff_index, batch, gnn1, gnn2, mlp1, mlp2):
    i32 = jnp.int32
    E = edge_index.shape[1]
    pad = E_PAD - E

    srcs = jnp.stack([
        jnp.concatenate([edge_index[0], jnp.full((pad,), N, i32)]),
        jnp.concatenate([diff_index[0], jnp.full((pad,), N, i32)]),
    ]).reshape(NC, NS, CT, CHUNK)
    dsts = jnp.stack([
        jnp.concatenate([edge_index[1], jnp.full((pad,), N, i32)]),
        jnp.concatenate([diff_index[1], jnp.full((pad,), N, i32)]),
    ]).reshape(NC, NS, CT, CHUNK)

    onehot = jnp.concatenate(
        [jnp.ones((CHUNK, 1), jnp.float32), jnp.zeros((CHUNK, 15), jnp.float32)],
        axis=1)
    zeros16 = jnp.zeros((N_PAD, 16), jnp.float32)
    zeros_hf = jnp.zeros((N_PAD, HF), jnp.float32)

    x_pad = jnp.concatenate([x, jnp.zeros((N_PAD - N, D), jnp.float32)])
    batch3 = jnp.concatenate([batch, jnp.full((N_PAD - N,), G, i32)]
                             ).reshape(NBLK, 1, 128)

    W1 = jnp.stack([gnn1['W0'], gnn2['W0']])
    b1 = jnp.stack([gnn1['b0'], gnn2['b0']])
    W2 = jnp.stack([gnn1['W1'], gnn2['W1']])
    b2 = jnp.stack([gnn1['b1'], gnn2['b1']])
    a1 = jnp.stack([gnn1['a0'], gnn2['a0']])
    mW = jnp.stack([mlp1['W0'], mlp1['W1'], mlp1['W2'], mlp1['Ws']])
    mb = jnp.stack([mlp1['b0'], mlp1['b1'], mlp1['b2'], mlp1['bs']])
    pscal = jnp.stack([gnn1['a1'], gnn2['a1'],
                       mlp1['a0'], mlp1['a1'], mlp1['a2'],
                       mlp2['a0'], mlp2['a1'], mlp2['a2']])
    gb = jnp.stack([mlp2['b0'], mlp2['b1'], mlp2['b2'], mlp2['bs']])
    gw12 = jnp.stack([mlp2['W1'], mlp2['W2']])

    cnt = _sc_degrees(dsts, onehot, zeros16)
    h1, g1lo, g1hi = _tc_pre(x_pad, cnt, W1, b1)
    agg1_lo, agg1_hi = _sc_conv(g1lo, g1hi, srcs, dsts, zeros_hf)
    h2, g2lo, g2hi, pool1 = _tc_mid(agg1_lo, agg1_hi, h1, cnt, batch3,
                                    W2, b2, a1)
    agg2_lo, agg2_hi = _sc_conv(g2lo, g2hi, srcs, dsts, zeros_hf)
    lv1, lv2, _, gv1, gv2 = _tc_post(agg2_lo, agg2_hi, h2, cnt, batch3,
                                     mW, mb, pscal,
                                     pool1, mlp2['W0'], gw12, mlp2['Ws'], gb)

    return (lv1[:N], gv1, lv2[:N], gv2)
